# baseline (speedup 1.0000x reference)
import functools

import jax
import jax.numpy as jnp
from jax import lax
from jax.experimental import pallas as pl
from jax.experimental.pallas import tpu as pltpu
from jax.experimental.pallas import tpu_sc as plsc

_MT = 2560
_MCC = 32
_NBUF = 2
_U = 8


def _tc_part_argmax(blk):
    m = blk.shape[0]
    mx = jnp.max(blk, axis=0)
    iota = lax.broadcasted_iota(jnp.int32, blk.shape, 0)
    idx = jnp.min(jnp.where(blk == mx[None, :], iota, m), axis=0)
    return mx, idx


def _tc_body(x1_ref, x2_ref, omx_ref, oix_ref):
    m1 = x1_ref.shape[1]
    mx1, idx1 = _tc_part_argmax(x1_ref[0])
    mx2, idx2 = _tc_part_argmax(x2_ref[0])
    first_low = mx1 >= mx2
    omx_ref[0, 0] = jnp.where(first_low, mx1, mx2)
    oix_ref[0, 0] = jnp.where(first_low, idx1, idx2 + m1)


def _tc_argmax(x, mt):
    B, M, N = x.shape
    MH = mt // 2
    mx, ix = pl.pallas_call(
        _tc_body,
        grid=(B,),
        in_specs=[
            pl.BlockSpec((1, MH, N), lambda b: (b, 0, 0)),
            pl.BlockSpec((1, MH, N), lambda b: (b, 1, 0)),
        ],
        out_specs=[
            pl.BlockSpec((1, 1, N), lambda b: (b, 0, 0)),
            pl.BlockSpec((1, 1, N), lambda b: (b, 0, 0)),
        ],
        out_shape=[
            jax.ShapeDtypeStruct((B, 1, N), jnp.float32),
            jax.ShapeDtypeStruct((B, 1, N), jnp.int32),
        ],
    )(x, x)
    return mx.reshape(B, N), ix.reshape(B, N)


def _sc_argmax(x, m_lo):
    B, M, N = x.shape
    seg = (M - m_lo) // 2
    nchunks = seg // _MCC
    ngroups = N // 16
    mesh = plsc.VectorSubcoreMesh(core_axis_name="c", subcore_axis_name="s")

    @functools.partial(
        pl.kernel,
        out_type=(
            jax.ShapeDtypeStruct((B * 2 * N,), jnp.float32),
            jax.ShapeDtypeStruct((B * 2 * N,), jnp.int32),
        ),
        mesh=mesh,
        scratch_types=[
            *[pltpu.VMEM((_MCC, N), jnp.float32) for _ in range(_NBUF)],
            pltpu.VMEM((N,), jnp.float32),
            pltpu.VMEM((N,), jnp.int32),
            *[pltpu.SemaphoreType.DMA for _ in range(_NBUF)],
        ],
    )
    def sc_kernel(x_hbm, omx_hbm, oix_hbm, buf0, buf1, mx_v, ix_v, sem0, sem1):
        bufs = (buf0, buf1)
        sems = (sem0, sem1)
        c = lax.axis_index("c")
        s = lax.axis_index("s")
        b = c * 8 + s // 2
        h = s % 2
        m0 = m_lo + h * seg

        def start(ck, j):
            pltpu.async_copy(
                x_hbm.at[b, pl.ds(m0 + ck * _MCC, _MCC), pl.ds(0, N)],
                bufs[j],
                sems[j],
            )

        neg = jnp.full((16,), -jnp.inf, jnp.float32)
        zer = jnp.zeros((16,), jnp.int32)

        def initg(g, _):
            mx_v[pl.ds(g * 16, 16)] = neg
            ix_v[pl.ds(g * 16, 16)] = zer
            return 0

        lax.fori_loop(0, ngroups, initg, 0)

        for j in range(_NBUF):
            start(j, j)

        def chunk_body(buf, base):
            def groupf(g, _, buf=buf):
                sl = pl.ds(g * 16, 16)
                ca = mx_v[sl]
                ia = ix_v[sl]
                cb = jnp.full((16,), -jnp.inf, jnp.float32)
                ib = jnp.zeros((16,), jnp.int32)

                def step(i, st, buf=buf, sl=sl):
                    sca, sia, scb, sib, mv = st
                    r0 = i * _U
                    for d in range(0, _U, 2):
                        va = buf[r0 + d, sl]
                        vb = buf[r0 + d + 1, sl]
                        ga = va > sca
                        gb = vb > scb
                        sca = jnp.where(ga, va, sca)
                        sia = jnp.where(ga, mv + d, sia)
                        scb = jnp.where(gb, vb, scb)
                        sib = jnp.where(gb, mv + (d + 1), sib)
                    return sca, sia, scb, sib, mv + _U

                mv0 = jnp.broadcast_to(base, (16,)).astype(jnp.int32)
                ca, ia, cb, ib, _mv = lax.fori_loop(
                    0, _MCC // _U, step, (ca, ia, cb, ib, mv0)
                )
                take_b = (cb > ca) | ((cb == ca) & (ib < ia))
                mx_v[sl] = jnp.where(take_b, cb, ca)
                ix_v[sl] = jnp.where(take_b, ib, ia)
                return 0

            lax.fori_loop(0, ngroups, groupf, 0)

        def outer(k, _):
            for j in range(_NBUF):
                ck = k * _NBUF + j
                pltpu.make_async_copy(
                    x_hbm.at[b, pl.ds(m0, _MCC), pl.ds(0, N)], bufs[j], sems[j]
                ).wait()
                chunk_body(bufs[j], m0 + ck * _MCC)

                @pl.when(ck + _NBUF < nchunks)
                def _prefetch(ck=ck, j=j):
                    start(ck + _NBUF, j)

            return 0

        lax.fori_loop(0, nchunks // _NBUF, outer, 0)

        off = (b * 2 + h) * N
        pltpu.sync_copy(mx_v, omx_hbm.at[pl.ds(off, N)])
        pltpu.sync_copy(ix_v, oix_hbm.at[pl.ds(off, N)])

    omx, oix = sc_kernel(x)
    return omx.reshape(B, 2, N), oix.reshape(B, 2, N)


def _merge_body(tmx_ref, tix_ref, smx_ref, six_ref, o_ref):
    s0mx = smx_ref[:, 0]
    s0ix = six_ref[:, 0]
    s1mx = smx_ref[:, 1]
    s1ix = six_ref[:, 1]
    lo = s0mx >= s1mx
    smx = jnp.where(lo, s0mx, s1mx)
    six = jnp.where(lo, s0ix, s1ix)
    tfirst = tmx_ref[...] >= smx
    o_ref[...] = jnp.where(tfirst, tix_ref[...], six)


def _merge(tmx, tix, smx, six):
    B, N = tmx.shape
    return pl.pallas_call(
        _merge_body,
        in_specs=[
            pl.BlockSpec((B, N), lambda: (0, 0)),
            pl.BlockSpec((B, N), lambda: (0, 0)),
            pl.BlockSpec((B, 2, N), lambda: (0, 0, 0)),
            pl.BlockSpec((B, 2, N), lambda: (0, 0, 0)),
        ],
        out_specs=pl.BlockSpec((B, N), lambda: (0, 0)),
        out_shape=jax.ShapeDtypeStruct((B, N), jnp.int32),
    )(tmx, tix, smx, six)


def kernel(x):
    B, M, N = x.shape
    smx, six = _sc_argmax(x, _MT)
    tmx, tix = _tc_argmax(x, _MT)
    return _merge(tmx, tix, smx, six)


# speedup vs baseline: 1.0580x; 1.0580x over previous
import functools

import jax
import jax.numpy as jnp
from jax import lax
from jax.experimental import pallas as pl
from jax.experimental.pallas import tpu as pltpu
from jax.experimental.pallas import tpu_sc as plsc

_NSC = 384
_MCC = 64
_NBUF = 2
_U = 8


def _tc_part_argmax(blk):
    m = blk.shape[0]
    mx = jnp.max(blk, axis=0)
    iota = lax.broadcasted_iota(jnp.int32, blk.shape, 0)
    idx = jnp.min(jnp.where(blk == mx[None, :], iota, m), axis=0)
    return mx, idx


def _tc_body(x1_ref, x2_ref, o_ref):
    m1 = x1_ref.shape[1]
    mx1, idx1 = _tc_part_argmax(x1_ref[0])
    mx2, idx2 = _tc_part_argmax(x2_ref[0])
    first_low = mx1 >= mx2
    o_ref[0, 0] = jnp.where(first_low, idx1, idx2 + m1)


def _tc_argmax(x, ntc):
    B, M, N = x.shape
    MH = M // 2
    out = pl.pallas_call(
        _tc_body,
        grid=(B,),
        in_specs=[
            pl.BlockSpec((1, MH, ntc), lambda b: (b, 0, 0)),
            pl.BlockSpec((1, MH, ntc), lambda b: (b, 1, 0)),
        ],
        out_specs=pl.BlockSpec((1, 1, ntc), lambda b: (b, 0, 0)),
        out_shape=jax.ShapeDtypeStruct((B, 1, ntc), jnp.int32),
    )(x, x)
    return out.reshape(B, ntc)


def _sc_argmax(x, nc0):
    B, M, N = x.shape
    nsc = N - nc0
    seg = M // 2
    nchunks = seg // _MCC
    ngroups = nsc // 16
    mesh = plsc.VectorSubcoreMesh(core_axis_name="c", subcore_axis_name="s")

    @functools.partial(
        pl.kernel,
        out_type=jax.ShapeDtypeStruct((B * nsc,), jnp.int32),
        mesh=mesh,
        scratch_types=[
            *[pltpu.VMEM((_MCC, nsc), jnp.float32) for _ in range(_NBUF)],
            pltpu.VMEM((nsc,), jnp.float32),
            pltpu.VMEM((nsc,), jnp.int32),
            pltpu.VMEM_SHARED((16, nsc), jnp.float32),
            pltpu.VMEM_SHARED((16, nsc), jnp.int32),
            pltpu.VMEM((2, nsc), jnp.float32),
            pltpu.VMEM((2, nsc), jnp.int32),
            pltpu.VMEM((nsc,), jnp.int32),
            *[pltpu.SemaphoreType.DMA for _ in range(_NBUF)],
            pltpu.SemaphoreType.DMA,
        ],
    )
    def sc_kernel(x_hbm, out_hbm, buf0, buf1, mx_v, ix_v, sh_mx, sh_ix,
                  tmp_mx, tmp_ix, res_ix, sem0, sem1, semm):
        bufs = (buf0, buf1)
        sems = (sem0, sem1)
        c = lax.axis_index("c")
        s = lax.axis_index("s")
        b = c * 8 + s // 2
        h = s % 2
        m0 = h * seg

        def start(ck, j):
            pltpu.async_copy(
                x_hbm.at[b, pl.ds(m0 + ck * _MCC, _MCC), pl.ds(nc0, nsc)],
                bufs[j],
                sems[j],
            )

        neg = jnp.full((16,), -jnp.inf, jnp.float32)
        zer = jnp.zeros((16,), jnp.int32)

        def initg(g, _):
            mx_v[pl.ds(g * 16, 16)] = neg
            ix_v[pl.ds(g * 16, 16)] = zer
            return 0

        lax.fori_loop(0, ngroups, initg, 0)

        for j in range(_NBUF):
            start(j, j)

        def chunk_body(buf, base):
            def groupf(g, _, buf=buf):
                sl = pl.ds(g * 16, 16)
                ca = mx_v[sl]
                ia = ix_v[sl]
                cb = jnp.full((16,), -jnp.inf, jnp.float32)
                ib = jnp.zeros((16,), jnp.int32)

                def step(i, st, buf=buf, sl=sl):
                    sca, sia, scb, sib, mv = st
                    r0 = i * _U
                    for d in range(0, _U, 2):
                        va = buf[r0 + d, sl]
                        vb = buf[r0 + d + 1, sl]
                        ga = va > sca
                        gb = vb > scb
                        sca = jnp.where(ga, va, sca)
                        sia = jnp.where(ga, mv + d, sia)
                        scb = jnp.where(gb, vb, scb)
                        sib = jnp.where(gb, mv + (d + 1), sib)
                    return sca, sia, scb, sib, mv + _U

                mv0 = jnp.broadcast_to(base, (16,)).astype(jnp.int32)
                ca, ia, cb, ib, _mv = lax.fori_loop(
                    0, _MCC // _U, step, (ca, ia, cb, ib, mv0)
                )
                take_b = (cb > ca) | ((cb == ca) & (ib < ia))
                mx_v[sl] = jnp.where(take_b, cb, ca)
                ix_v[sl] = jnp.where(take_b, ib, ia)
                return 0

            lax.fori_loop(0, ngroups, groupf, 0)

        def outer(k, _):
            for j in range(_NBUF):
                ck = k * _NBUF + j
                pltpu.make_async_copy(
                    x_hbm.at[b, pl.ds(m0, _MCC), pl.ds(nc0, nsc)],
                    bufs[j], sems[j],
                ).wait()
                chunk_body(bufs[j], m0 + ck * _MCC)

                @pl.when(ck + _NBUF < nchunks)
                def _prefetch(ck=ck, j=j):
                    start(ck + _NBUF, j)

            return 0

        lax.fori_loop(0, nchunks // _NBUF, outer, 0)

        pltpu.sync_copy(mx_v, sh_mx.at[s])
        pltpu.sync_copy(ix_v, sh_ix.at[s])
        plsc.subcore_barrier()

        @pl.when(h == 0)
        def _merge():
            pltpu.async_copy(sh_mx.at[s], tmp_mx.at[0], semm).wait()
            pltpu.async_copy(sh_mx.at[s + 1], tmp_mx.at[1], semm).wait()
            pltpu.async_copy(sh_ix.at[s], tmp_ix.at[0], semm).wait()
            pltpu.async_copy(sh_ix.at[s + 1], tmp_ix.at[1], semm).wait()

            def mergef(g, _):
                sl = pl.ds(g * 16, 16)
                cm = tmp_mx[0, sl]
                ci = tmp_ix[0, sl]
                vm = tmp_mx[1, sl]
                vi = tmp_ix[1, sl]
                gt = vm > cm
                res_ix[sl] = jnp.where(gt, vi, ci)
                return 0

            lax.fori_loop(0, ngroups, mergef, 0)
            pltpu.sync_copy(res_ix, out_hbm.at[pl.ds(b * nsc, nsc)])

    return sc_kernel(x).reshape(B, nsc)


def kernel(x):
    B, M, N = x.shape
    ntc = N - _NSC
    out_sc = _sc_argmax(x, ntc)
    out_tc = _tc_argmax(x, ntc)
    return jnp.concatenate([out_tc, out_sc], axis=1)
